# Initial kernel scaffold; baseline (speedup 1.0000x reference)
import jax
import jax.numpy as jnp
from jax.experimental import pallas as pl


def kernel(x, A):
    raise NotImplementedError("write your pallas kernel here")


# speedup vs baseline: 1.0971x; 1.0971x over previous
import functools

import jax
import jax.numpy as jnp
from jax.experimental import pallas as pl
from jax.experimental.pallas import tpu as pltpu

_GAMMA = 1.0
_LAM = 1.0
_N_ITER = 4
_LR = 0.02
_EPS = 1e-3
_BISECT_ITERS = 30
_JT = 512


def _fusedmax_body(xr_ref, xc_ref, a_ref, o_ref):
    x_row = xr_ref[0]
    x_col = xc_ref[0]
    y_row = x_row
    y_col = x_col
    M = x_row.shape[1]
    n_tiles = M // _JT
    for _ in range(_N_ITER):
        pen_col = jnp.zeros_like(y_col)
        pen_row_tiles = []
        for t in range(n_tiles):
            a_t = a_ref[0, :, t * _JT:(t + 1) * _JT]
            d = y_col - y_row[:, t * _JT:(t + 1) * _JT]
            p = a_t * jnp.tanh(d * (1.0 / _EPS))
            pen_col = pen_col + jnp.sum(p, axis=1, keepdims=True)
            pen_row_tiles.append(-jnp.sum(p, axis=0, keepdims=True))
        pen_row = jnp.concatenate(pen_row_tiles, axis=1)
        y_col = y_col - _LR * ((y_col - x_col) + _LAM * pen_col)
        y_row = y_row - _LR * ((y_row - x_row) + _LAM * pen_row)

    z = y_row * (1.0 / _GAMMA)
    zmax = jnp.max(z)
    lo = zmax - 1.0
    hi = zmax

    def bis(_, carry):
        lo, hi = carry
        mid = 0.5 * (lo + hi)
        f = jnp.sum(jnp.maximum(z - mid, 0.0))
        gt = f > 1.0
        return jnp.where(gt, mid, lo), jnp.where(gt, hi, mid)

    lo, hi = jax.lax.fori_loop(0, _BISECT_ITERS, bis, (lo, hi))
    tau = 0.5 * (lo + hi)
    o_ref[0] = jnp.maximum(y_row - tau, 0.0)


@jax.jit
def kernel(x, A):
    B, M = x.shape
    x3 = x.reshape(B, 1, M)
    xt3 = x.reshape(B, M, 1)
    grid_spec = pl.GridSpec(
        grid=(B,),
        in_specs=[
            pl.BlockSpec((1, 1, M), lambda b: (b, 0, 0)),
            pl.BlockSpec((1, M, 1), lambda b: (b, 0, 0)),
            pl.BlockSpec((1, M, M), lambda b: (b, 0, 0)),
        ],
        out_specs=pl.BlockSpec((1, 1, M), lambda b: (b, 0, 0)),
    )
    out = pl.pallas_call(
        _fusedmax_body,
        grid_spec=grid_spec,
        out_shape=jax.ShapeDtypeStruct((B, 1, M), x.dtype),
    )(x3, xt3, A)
    return out.reshape(B, M)


# speedup vs baseline: 1.2424x; 1.1325x over previous
import functools

import jax
import jax.numpy as jnp
from jax.experimental import pallas as pl
from jax.experimental.pallas import tpu as pltpu

_GAMMA = 1.0
_LAM = 1.0
_N_ITER = 4
_LR = 0.02
_EPS = 1e-3
_BISECT_ITERS = 30
_JT = 256


def _fusedmax_body(xc_ref, a_ref, o_ref):
    x_col = xc_ref[0]
    y = x_col
    M = x_col.shape[0]
    nt = M // _JT
    r_io = jax.lax.broadcasted_iota(jnp.int32, (_JT, _JT), 0)
    c_io = jax.lax.broadcasted_iota(jnp.int32, (_JT, _JT), 1)
    triu = r_io < c_io

    def sl(t):
        return slice(t * _JT, (t + 1) * _JT)

    for _ in range(_N_ITER):
        u = y * (1.0 / _EPS)
        ut = u.reshape(1, M) if False else jnp.transpose(u)
        rs = [None] * nt
        cs = [None] * nt
        for ti in range(nt):
            for tj in range(ti, nt):
                a_t = a_ref[0, sl(ti), sl(tj)]
                d = u[sl(ti)] - ut[:, sl(tj)]
                p = a_t * jnp.tanh(d)
                if ti == tj:
                    p = jnp.where(triu, p, 0.0)
                prs = jnp.sum(p, axis=1, keepdims=True)
                pcs = jnp.sum(p, axis=0, keepdims=True)
                rs[ti] = prs if rs[ti] is None else rs[ti] + prs
                cs[tj] = pcs if cs[tj] is None else cs[tj] + pcs
        rsv = jnp.concatenate(rs, axis=0)
        csv = jnp.concatenate(cs, axis=1)
        pen = rsv - jnp.transpose(csv)
        y = y - _LR * ((y - x_col) + _LAM * pen)

    z = jnp.transpose(y) * (1.0 / _GAMMA)
    zmax = jnp.max(z)
    lo = zmax - 1.0
    hi = zmax

    def bis(_, carry):
        lo, hi = carry
        mid = 0.5 * (lo + hi)
        f = jnp.sum(jnp.maximum(z - mid, 0.0))
        gt = f > 1.0
        return jnp.where(gt, mid, lo), jnp.where(gt, hi, mid)

    lo, hi = jax.lax.fori_loop(0, _BISECT_ITERS, bis, (lo, hi))
    tau = 0.5 * (lo + hi)
    o_ref[0] = jnp.maximum(jnp.transpose(y) - tau, 0.0)


@jax.jit
def kernel(x, A):
    B, M = x.shape
    xt3 = x.reshape(B, M, 1)
    grid_spec = pl.GridSpec(
        grid=(B,),
        in_specs=[
            pl.BlockSpec((1, M, 1), lambda b: (b, 0, 0)),
            pl.BlockSpec((1, M, M), lambda b: (b, 0, 0)),
        ],
        out_specs=pl.BlockSpec((1, 1, M), lambda b: (b, 0, 0)),
    )
    out = pl.pallas_call(
        _fusedmax_body,
        grid_spec=grid_spec,
        out_shape=jax.ShapeDtypeStruct((B, 1, M), x.dtype),
    )(xt3, A)
    return out.reshape(B, M)


# speedup vs baseline: 1.3148x; 1.0582x over previous
import functools

import jax
import jax.numpy as jnp
from jax.experimental import pallas as pl
from jax.experimental.pallas import tpu as pltpu

_GAMMA = 1.0
_LAM = 1.0
_N_ITER = 4
_LR = 0.02
_EPS = 1e-3
_BISECT_ITERS = 30
_NB = 4
_JT = 512


def _fusedmax_body(xc_ref, *refs):
    a_bands = refs[:_NB]
    o_ref = refs[_NB]
    x_col = xc_ref[0]
    y = x_col
    M = x_col.shape[0]
    nt = M // _JT
    r_io = jax.lax.broadcasted_iota(jnp.int32, (_JT, _JT), 0)
    c_io = jax.lax.broadcasted_iota(jnp.int32, (_JT, _JT), 1)
    triu = r_io < c_io

    def sl(t):
        return slice(t * _JT, (t + 1) * _JT)

    for _ in range(_N_ITER):
        u = y * (1.0 / _EPS)
        ut = jnp.transpose(u)
        rs = [None] * nt
        cs = [None] * nt
        for ti in range(nt):
            for tj in range(ti, nt):
                a_t = a_bands[ti][0, 0][:, sl(tj)]
                d = u[sl(ti)] - ut[:, sl(tj)]
                p = a_t * jnp.tanh(d)
                if ti == tj:
                    p = jnp.where(triu, p, 0.0)
                prs = jnp.sum(p, axis=1, keepdims=True)
                pcs = jnp.sum(p, axis=0, keepdims=True)
                rs[ti] = prs if rs[ti] is None else rs[ti] + prs
                cs[tj] = pcs if cs[tj] is None else cs[tj] + pcs
        rsv = jnp.concatenate(rs, axis=0)
        csv = jnp.concatenate(cs, axis=1)
        pen = rsv - jnp.transpose(csv)
        y = y - _LR * ((y - x_col) + _LAM * pen)

    z = jnp.transpose(y) * (1.0 / _GAMMA)
    zmax = jnp.max(z)
    lo = zmax - 1.0
    hi = zmax

    def bis(_, carry):
        lo, hi = carry
        mid = 0.5 * (lo + hi)
        f = jnp.sum(jnp.maximum(z - mid, 0.0))
        gt = f > 1.0
        return jnp.where(gt, mid, lo), jnp.where(gt, hi, mid)

    lo, hi = jax.lax.fori_loop(0, _BISECT_ITERS, bis, (lo, hi))
    tau = 0.5 * (lo + hi)
    o_ref[0] = jnp.maximum(jnp.transpose(y) - tau, 0.0)


@jax.jit
def kernel(x, A):
    B, M = x.shape
    xt3 = x.reshape(B, M, 1)
    A4 = A.reshape(B, _NB, M // _NB, M)

    def band_spec(q):
        return pl.BlockSpec((1, 1, M // _NB, M), lambda b, q=q: (b, q, 0, 0))

    grid_spec = pl.GridSpec(
        grid=(B,),
        in_specs=[pl.BlockSpec((1, M, 1), lambda b: (b, 0, 0))]
        + [band_spec(q) for q in range(_NB)],
        out_specs=pl.BlockSpec((1, 1, M), lambda b: (b, 0, 0)),
    )
    out = pl.pallas_call(
        _fusedmax_body,
        grid_spec=grid_spec,
        out_shape=jax.ShapeDtypeStruct((B, 1, M), x.dtype),
    )(xt3, *([A4] * _NB))
    return out.reshape(B, M)
